# baseline (speedup 1.0000x reference)
import jax
import jax.numpy as jnp
from jax import lax
from jax.experimental import pallas as pl
from jax.experimental.pallas import tpu as pltpu
from jax.experimental.pallas import tpu_sc as plsc

NC = 2
NS = 16
NW = NC * NS
DEG_W = 16
K = 128
NB = 4


def _zero_rows(ref, nrows, ncols):
    zeros16 = jnp.zeros((16,), jnp.float32)

    def body(i, carry):
        for c in range(ncols // 16):
            ref[i, pl.ds(c * 16, 16)] = zeros16
        return carry

    lax.fori_loop(0, nrows, body, 0)


def _deg_body(refB_hbm, degp_hbm, idx_v, ones_v, zbuf_v, deg_sp):
    cid = lax.axis_index("c")
    sid = lax.axis_index("s")
    n_pad = deg_sp.shape[0]
    stripe = n_pad // NS
    n_chunks = idx_v.shape[0]

    ones16 = jnp.ones((16,), jnp.float32)

    def fill_ones(i, carry):
        ones_v[i, :] = ones16
        return carry

    lax.fori_loop(0, ones_v.shape[0], fill_ones, 0)
    _zero_rows(zbuf_v, zbuf_v.shape[0], DEG_W)

    pltpu.sync_copy(zbuf_v, deg_sp.at[pl.ds(sid * stripe, stripe)])
    plsc.subcore_barrier()

    wid = cid * NS + sid
    pltpu.sync_copy(refB_hbm.at[wid], idx_v)

    def chunk(j, carry):
        pltpu.sync_copy(ones_v, deg_sp.at[idx_v.at[j]], add=True)
        return carry

    lax.fori_loop(0, n_chunks, chunk, 0)
    plsc.subcore_barrier()

    pltpu.sync_copy(
        deg_sp.at[pl.ds(sid * stripe, stripe)],
        degp_hbm.at[cid, pl.ds(sid * stripe, stripe)],
    )


def _norm_body(degp_ref, x_ref, xn0_ref, xn1_ref):
    deg = degp_ref[0, :, 0:1] + degp_ref[1, :, 0:1]
    r = jnp.where(deg > 0, lax.rsqrt(jnp.maximum(deg, 1.0)), 1.0)
    x = x_ref[...]
    xn = x * r
    h = x.shape[1] // 2
    xn0_ref[...] = xn[:, :h]
    xn1_ref[...] = xn[:, h:]


def _conv_body(xn0_hbm, xn1_hbm, refA_hbm, refB_hbm, p0_hbm, p1_hbm,
               idxa_v, idxb_v, rows_v, zbuf_v, acc_sp, semg, sems):
    cid = lax.axis_index("c")
    sid = lax.axis_index("s")
    n_pad = acc_sp.shape[0]
    dh = acc_sp.shape[1]
    stripe = n_pad // NS
    zrows = zbuf_v.shape[0]
    n_chunks = idxa_v.shape[0]

    _zero_rows(zbuf_v, zrows, dh)

    wid = cid * NS + sid
    pltpu.sync_copy(refA_hbm.at[wid], idxa_v)
    pltpu.sync_copy(refB_hbm.at[wid], idxb_v)

    for h, (xn_hbm, p_hbm) in enumerate(((xn0_hbm, p0_hbm), (xn1_hbm, p1_hbm))):
        for j in range(stripe // zrows):
            pltpu.sync_copy(
                zbuf_v, acc_sp.at[pl.ds(sid * stripe + j * zrows, zrows)])
        plsc.subcore_barrier()

        def visit(i0, carry):
            gathers = []
            for b in range(NB):
                j = i0 * NB + b
                gathers.append(pltpu.async_copy(
                    xn_hbm.at[idxa_v.at[j]], rows_v.at[b], semg.at[b]))
            for b in range(NB):
                j = i0 * NB + b
                gathers[b].wait()
                pltpu.sync_copy(
                    rows_v.at[b], acc_sp.at[idxb_v.at[j]], add=True)
            return carry

        lax.fori_loop(0, n_chunks // NB, visit, 0)
        plsc.subcore_barrier()

        pltpu.sync_copy(
            acc_sp.at[pl.ds(sid * stripe, stripe)],
            p_hbm.at[cid, pl.ds(sid * stripe, stripe)],
        )


def _out_body(x_ref, p0_ref, p1_ref, degp_ref, w_ref, b_ref, o_ref):
    deg = degp_ref[0, :, 0:1] + degp_ref[1, :, 0:1]
    r = jnp.where(deg > 0, lax.rsqrt(jnp.maximum(deg, 1.0)), 1.0)
    conv = jnp.concatenate(
        [p0_ref[0] + p0_ref[1], p1_ref[0] + p1_ref[1]], axis=-1)
    s = r * (x_ref[...] + conv)
    y = jnp.dot(s, w_ref[...], preferred_element_type=jnp.float32) + b_ref[...]
    o_ref[...] = jnp.maximum(y, 0.0)


def kernel(X, ref_A, ref_B, W, b):
    N, D = X.shape
    U = W.shape[1]
    E = ref_A.shape[0]
    DH = D // 2
    e_tile = ((E // NW + K * NB - 1) // (K * NB)) * (K * NB)
    n_chunks = e_tile // K
    e_pad = NW * e_tile - E
    n_pad = ((N + NS * DEG_W - 1) // (NS * DEG_W)) * (NS * DEG_W)

    refA3 = jnp.concatenate(
        [ref_A.astype(jnp.int32), jnp.zeros((e_pad,), jnp.int32)]
    ).reshape(NW, n_chunks, K)
    refB3 = jnp.concatenate(
        [ref_B.astype(jnp.int32), jnp.full((e_pad,), N, jnp.int32)]
    ).reshape(NW, n_chunks, K)

    mesh = plsc.VectorSubcoreMesh(core_axis_name="c", subcore_axis_name="s")

    degp = pl.kernel(
        _deg_body,
        out_type=jax.ShapeDtypeStruct((NC, n_pad, DEG_W), jnp.float32),
        mesh=mesh,
        compiler_params=pltpu.CompilerParams(use_tc_tiling_on_sc=False),
        scratch_types=[
            pltpu.VMEM((n_chunks, K), jnp.int32),
            pltpu.VMEM((K, DEG_W), jnp.float32),
            pltpu.VMEM((n_pad // NS, DEG_W), jnp.float32),
            pltpu.VMEM_SHARED((n_pad, DEG_W), jnp.float32),
        ],
    )(refB3)

    grid = 10
    br = N // grid
    xn0, xn1 = pl.pallas_call(
        _norm_body,
        grid=(grid,),
        in_specs=[
            pl.BlockSpec((NC, br, DEG_W), lambda i: (0, i, 0)),
            pl.BlockSpec((br, D), lambda i: (i, 0)),
        ],
        out_specs=[
            pl.BlockSpec((br, DH), lambda i: (i, 0)),
            pl.BlockSpec((br, DH), lambda i: (i, 0)),
        ],
        out_shape=[
            jax.ShapeDtypeStruct((N, DH), jnp.float32),
            jax.ShapeDtypeStruct((N, DH), jnp.float32),
        ],
    )(degp, X)

    p0, p1 = pl.kernel(
        _conv_body,
        out_type=[
            jax.ShapeDtypeStruct((NC, n_pad, DH), jnp.float32),
            jax.ShapeDtypeStruct((NC, n_pad, DH), jnp.float32),
        ],
        mesh=mesh,
        compiler_params=pltpu.CompilerParams(use_tc_tiling_on_sc=False),
        scratch_types=[
            pltpu.VMEM((n_chunks, K), jnp.int32),
            pltpu.VMEM((n_chunks, K), jnp.int32),
            pltpu.VMEM((NB, K, DH), jnp.float32),
            pltpu.VMEM((128, DH), jnp.float32),
            pltpu.VMEM_SHARED((n_pad, DH), jnp.float32),
            pltpu.SemaphoreType.DMA((NB,)),
            pltpu.SemaphoreType.DMA((NB,)),
        ],
    )(xn0, xn1, refA3, refB3)

    b2 = b.reshape(1, U)
    out = pl.pallas_call(
        _out_body,
        grid=(grid,),
        in_specs=[
            pl.BlockSpec((br, D), lambda i: (i, 0)),
            pl.BlockSpec((NC, br, DH), lambda i: (0, i, 0)),
            pl.BlockSpec((NC, br, DH), lambda i: (0, i, 0)),
            pl.BlockSpec((NC, br, DEG_W), lambda i: (0, i, 0)),
            pl.BlockSpec((D, U), lambda i: (0, 0)),
            pl.BlockSpec((1, U), lambda i: (0, 0)),
        ],
        out_specs=pl.BlockSpec((br, U), lambda i: (i, 0)),
        out_shape=jax.ShapeDtypeStruct((N, U), jnp.float32),
    )(X, p0, p1, degp, W, b2)
    return out


# speedup vs baseline: 1.0079x; 1.0079x over previous
import jax
import jax.numpy as jnp
from jax import lax
from jax.experimental import pallas as pl
from jax.experimental.pallas import tpu as pltpu
from jax.experimental.pallas import tpu_sc as plsc

NC = 2
NS = 16
NW = NC * NS
DEG_W = 16
K = 128
NB = 4


def _zero_rows(ref, nrows, ncols):
    zeros16 = jnp.zeros((16,), jnp.float32)

    def body(i, carry):
        for c in range(ncols // 16):
            ref[i, pl.ds(c * 16, 16)] = zeros16
        return carry

    lax.fori_loop(0, nrows, body, 0)


def _deg_body(refB_hbm, degp_hbm, idx_v, ones_v, zbuf_v, deg_sp):
    cid = lax.axis_index("c")
    sid = lax.axis_index("s")
    n_pad = deg_sp.shape[0]
    stripe = n_pad // NS
    n_chunks = idx_v.shape[0]

    ones16 = jnp.ones((16,), jnp.float32)

    def fill_ones(i, carry):
        ones_v[i, :] = ones16
        return carry

    lax.fori_loop(0, ones_v.shape[0], fill_ones, 0)
    _zero_rows(zbuf_v, zbuf_v.shape[0], DEG_W)

    pltpu.sync_copy(zbuf_v, deg_sp.at[pl.ds(sid * stripe, stripe)])
    plsc.subcore_barrier()

    wid = cid * NS + sid
    pltpu.sync_copy(refB_hbm.at[wid], idx_v)

    def chunk(j, carry):
        pltpu.sync_copy(ones_v, deg_sp.at[idx_v.at[j]], add=True)
        return carry

    lax.fori_loop(0, n_chunks, chunk, 0)
    plsc.subcore_barrier()

    pltpu.sync_copy(
        deg_sp.at[pl.ds(sid * stripe, stripe)],
        degp_hbm.at[cid, pl.ds(sid * stripe, stripe)],
    )


def _norm_body(degp_ref, x_ref, xn0_ref, xn1_ref):
    deg = degp_ref[0, :, 0:1] + degp_ref[1, :, 0:1]
    r = jnp.where(deg > 0, lax.rsqrt(jnp.maximum(deg, 1.0)), 1.0)
    x = x_ref[...]
    xn = x * r
    h = x.shape[1] // 2
    xn0_ref[...] = xn[:, :h]
    xn1_ref[...] = xn[:, h:]


def _conv_body(xn0_hbm, xn1_hbm, refA_hbm, refB_hbm, p0_hbm, p1_hbm,
               idxa_v, idxb_v, rows_v, zbuf_v, acc_sp, semg, sems):
    cid = lax.axis_index("c")
    sid = lax.axis_index("s")
    n_pad = acc_sp.shape[0]
    dh = acc_sp.shape[1]
    stripe = n_pad // NS
    zrows = zbuf_v.shape[0]
    n_chunks = idxa_v.shape[0]

    _zero_rows(zbuf_v, zrows, dh)

    wid = cid * NS + sid
    pltpu.sync_copy(refA_hbm.at[wid], idxa_v)
    pltpu.sync_copy(refB_hbm.at[wid], idxb_v)

    for h, (xn_hbm, p_hbm) in enumerate(((xn0_hbm, p0_hbm), (xn1_hbm, p1_hbm))):
        for j in range(stripe // zrows):
            pltpu.sync_copy(
                zbuf_v, acc_sp.at[pl.ds(sid * stripe + j * zrows, zrows)])
        plsc.subcore_barrier()

        def visit(i0, carry):
            gathers = []
            for b in range(NB):
                j = i0 * NB + b
                gathers.append(pltpu.async_copy(
                    xn_hbm.at[idxa_v.at[j]], rows_v.at[b], semg.at[b]))
            for b in range(NB):
                j = i0 * NB + b
                gathers[b].wait()
                pltpu.sync_copy(
                    rows_v.at[b], acc_sp.at[idxb_v.at[j]], add=True)
            return carry

        lax.fori_loop(0, n_chunks // NB, visit, 0)
        plsc.subcore_barrier()

        pltpu.sync_copy(
            acc_sp.at[pl.ds(sid * stripe, stripe)],
            p_hbm.at[cid, pl.ds(sid * stripe, stripe)],
        )


def _out_body(x_ref, p0_ref, p1_ref, degp_ref, w_ref, b_ref, o_ref):
    deg = degp_ref[0, :, 0:1] + degp_ref[1, :, 0:1]
    r = jnp.where(deg > 0, lax.rsqrt(jnp.maximum(deg, 1.0)), 1.0)
    conv = jnp.concatenate(
        [p0_ref[0] + p0_ref[1], p1_ref[0] + p1_ref[1]], axis=-1)
    s = r * (x_ref[...] + conv)
    y = jnp.dot(s, w_ref[...], preferred_element_type=jnp.float32) + b_ref[...]
    o_ref[...] = jnp.maximum(y, 0.0)


def kernel(X, ref_A, ref_B, W, b):
    N, D = X.shape
    U = W.shape[1]
    E = ref_A.shape[0]
    DH = D // 2
    e_tile = ((E // NW + K * NB - 1) // (K * NB)) * (K * NB)
    n_chunks = e_tile // K
    e_pad = NW * e_tile - E
    n_pad = ((N + NS * DEG_W - 1) // (NS * DEG_W)) * (NS * DEG_W)

    refA3 = jnp.concatenate(
        [ref_A.astype(jnp.int32), jnp.zeros((e_pad,), jnp.int32)]
    ).reshape(NW, n_chunks, K)
    pad_dst = N + (jnp.arange(e_pad, dtype=jnp.int32) % (n_pad - N))
    refB3 = jnp.concatenate(
        [ref_B.astype(jnp.int32), pad_dst]
    ).reshape(NW, n_chunks, K)

    mesh = plsc.VectorSubcoreMesh(core_axis_name="c", subcore_axis_name="s")

    degp = pl.kernel(
        _deg_body,
        out_type=jax.ShapeDtypeStruct((NC, n_pad, DEG_W), jnp.float32),
        mesh=mesh,
        compiler_params=pltpu.CompilerParams(use_tc_tiling_on_sc=False),
        scratch_types=[
            pltpu.VMEM((n_chunks, K), jnp.int32),
            pltpu.VMEM((K, DEG_W), jnp.float32),
            pltpu.VMEM((n_pad // NS, DEG_W), jnp.float32),
            pltpu.VMEM_SHARED((n_pad, DEG_W), jnp.float32),
        ],
    )(refB3)

    grid = 10
    br = N // grid
    xn0, xn1 = pl.pallas_call(
        _norm_body,
        grid=(grid,),
        in_specs=[
            pl.BlockSpec((NC, br, DEG_W), lambda i: (0, i, 0)),
            pl.BlockSpec((br, D), lambda i: (i, 0)),
        ],
        out_specs=[
            pl.BlockSpec((br, DH), lambda i: (i, 0)),
            pl.BlockSpec((br, DH), lambda i: (i, 0)),
        ],
        out_shape=[
            jax.ShapeDtypeStruct((N, DH), jnp.float32),
            jax.ShapeDtypeStruct((N, DH), jnp.float32),
        ],
    )(degp, X)

    p0, p1 = pl.kernel(
        _conv_body,
        out_type=[
            jax.ShapeDtypeStruct((NC, n_pad, DH), jnp.float32),
            jax.ShapeDtypeStruct((NC, n_pad, DH), jnp.float32),
        ],
        mesh=mesh,
        compiler_params=pltpu.CompilerParams(use_tc_tiling_on_sc=False),
        scratch_types=[
            pltpu.VMEM((n_chunks, K), jnp.int32),
            pltpu.VMEM((n_chunks, K), jnp.int32),
            pltpu.VMEM((NB, K, DH), jnp.float32),
            pltpu.VMEM((128, DH), jnp.float32),
            pltpu.VMEM_SHARED((n_pad, DH), jnp.float32),
            pltpu.SemaphoreType.DMA((NB,)),
            pltpu.SemaphoreType.DMA((NB,)),
        ],
    )(xn0, xn1, refA3, refB3)

    b2 = b.reshape(1, U)
    out = pl.pallas_call(
        _out_body,
        grid=(grid,),
        in_specs=[
            pl.BlockSpec((br, D), lambda i: (i, 0)),
            pl.BlockSpec((NC, br, DH), lambda i: (0, i, 0)),
            pl.BlockSpec((NC, br, DH), lambda i: (0, i, 0)),
            pl.BlockSpec((NC, br, DEG_W), lambda i: (0, i, 0)),
            pl.BlockSpec((D, U), lambda i: (0, 0)),
            pl.BlockSpec((1, U), lambda i: (0, 0)),
        ],
        out_specs=pl.BlockSpec((br, U), lambda i: (i, 0)),
        out_shape=jax.ShapeDtypeStruct((N, U), jnp.float32),
    )(X, p0, p1, degp, W, b2)
    return out


# speedup vs baseline: 1.6417x; 1.6288x over previous
import jax
import jax.numpy as jnp
from jax import lax
from jax.experimental import pallas as pl
from jax.experimental.pallas import tpu as pltpu
from jax.experimental.pallas import tpu_sc as plsc

NC = 2
NS = 16
NW = NC * NS
DEG_W = 16
K = 80
NB = 1


def _zero_rows(ref, nrows, ncols):
    zeros16 = jnp.zeros((16,), jnp.float32)

    def body(i, carry):
        for c in range(ncols // 16):
            ref[i, pl.ds(c * 16, 16)] = zeros16
        return carry

    lax.fori_loop(0, nrows, body, 0)


def _deg_body(refB_hbm, degp_hbm, idx_v, ones_v, zbuf_v, deg_sp):
    cid = lax.axis_index("c")
    sid = lax.axis_index("s")
    n_pad = deg_sp.shape[0]
    stripe = n_pad // NS
    n_chunks = idx_v.shape[0]

    ones16 = jnp.ones((16,), jnp.float32)

    def fill_ones(i, carry):
        ones_v[i, :] = ones16
        return carry

    lax.fori_loop(0, ones_v.shape[0], fill_ones, 0)
    _zero_rows(zbuf_v, zbuf_v.shape[0], DEG_W)

    pltpu.sync_copy(zbuf_v, deg_sp.at[pl.ds(sid * stripe, stripe)])
    plsc.subcore_barrier()

    wid = cid * NS + sid
    pltpu.sync_copy(refB_hbm.at[wid], idx_v)

    def chunk(j, carry):
        pltpu.sync_copy(ones_v, deg_sp.at[idx_v.at[j]], add=True)
        return carry

    lax.fori_loop(0, n_chunks, chunk, 0)
    plsc.subcore_barrier()

    pltpu.sync_copy(
        deg_sp.at[pl.ds(sid * stripe, stripe)],
        degp_hbm.at[cid, pl.ds(sid * stripe, stripe)],
    )


def _norm_body(degp_ref, x_ref, xn0_ref, xn1_ref):
    deg = degp_ref[0, :, 0:1] + degp_ref[1, :, 0:1]
    r = jnp.where(deg > 0, lax.rsqrt(jnp.maximum(deg, 1.0)), 1.0)
    x = x_ref[...]
    xn = x * r
    h = x.shape[1] // 2
    xn0_ref[...] = xn[:, :h]
    xn1_ref[...] = xn[:, h:]


def _conv_body(xn0_hbm, xn1_hbm, refA_hbm, refB_hbm, p0_hbm, p1_hbm,
               idxa_v, idxb_v, rows_v, zbuf_v, acc_sp, semg, sems):
    cid = lax.axis_index("c")
    sid = lax.axis_index("s")
    n_pad = acc_sp.shape[0]
    dh = acc_sp.shape[1]
    stripe = n_pad // NS
    zrows = zbuf_v.shape[0]
    n_chunks = idxa_v.shape[0]

    _zero_rows(zbuf_v, zrows, dh)

    wid = cid * NS + sid
    pltpu.sync_copy(refA_hbm.at[wid], idxa_v)
    pltpu.sync_copy(refB_hbm.at[wid], idxb_v)

    for h, (xn_hbm, p_hbm) in enumerate(((xn0_hbm, p0_hbm), (xn1_hbm, p1_hbm))):
        for j in range(stripe // zrows):
            pltpu.sync_copy(
                zbuf_v, acc_sp.at[pl.ds(sid * stripe + j * zrows, zrows)])
        plsc.subcore_barrier()

        def visit(i0, carry):
            gathers = []
            for b in range(NB):
                j = i0 * NB + b
                gathers.append(pltpu.async_copy(
                    xn_hbm.at[idxa_v.at[j]], rows_v.at[b], semg.at[b]))
            for b in range(NB):
                j = i0 * NB + b
                gathers[b].wait()
                pltpu.sync_copy(
                    rows_v.at[b], acc_sp.at[idxb_v.at[j]], add=True)
            return carry

        lax.fori_loop(0, n_chunks // NB, visit, 0)
        plsc.subcore_barrier()

        pltpu.sync_copy(
            acc_sp.at[pl.ds(sid * stripe, stripe)],
            p_hbm.at[cid, pl.ds(sid * stripe, stripe)],
        )


def _out_body(x_ref, p0_ref, p1_ref, degp_ref, w_ref, b_ref, o_ref):
    deg = degp_ref[0, :, 0:1] + degp_ref[1, :, 0:1]
    r = jnp.where(deg > 0, lax.rsqrt(jnp.maximum(deg, 1.0)), 1.0)
    conv = jnp.concatenate(
        [p0_ref[0] + p0_ref[1], p1_ref[0] + p1_ref[1]], axis=-1)
    s = r * (x_ref[...] + conv)
    y = jnp.dot(s, w_ref[...], preferred_element_type=jnp.float32) + b_ref[...]
    o_ref[...] = jnp.maximum(y, 0.0)


def kernel(X, ref_A, ref_B, W, b):
    N, D = X.shape
    U = W.shape[1]
    E = ref_A.shape[0]
    DH = D // 2
    e_tile = ((E // NW + K * NB - 1) // (K * NB)) * (K * NB)
    n_chunks = e_tile // K
    e_pad = NW * e_tile - E
    n_pad = ((N + NS * DEG_W - 1) // (NS * DEG_W)) * (NS * DEG_W)

    refA3 = jnp.concatenate(
        [ref_A.astype(jnp.int32), jnp.zeros((e_pad,), jnp.int32)]
    ).reshape(NW, n_chunks, K)
    pad_dst = N + (jnp.arange(e_pad, dtype=jnp.int32) % (n_pad - N))
    refB3 = jnp.concatenate(
        [ref_B.astype(jnp.int32), pad_dst]
    ).reshape(NW, n_chunks, K)

    mesh = plsc.VectorSubcoreMesh(core_axis_name="c", subcore_axis_name="s")

    degp = pl.kernel(
        _deg_body,
        out_type=jax.ShapeDtypeStruct((NC, n_pad, DEG_W), jnp.float32),
        mesh=mesh,
        compiler_params=pltpu.CompilerParams(use_tc_tiling_on_sc=False),
        scratch_types=[
            pltpu.VMEM((n_chunks, K), jnp.int32),
            pltpu.VMEM((K, DEG_W), jnp.float32),
            pltpu.VMEM((n_pad // NS, DEG_W), jnp.float32),
            pltpu.VMEM_SHARED((n_pad, DEG_W), jnp.float32),
        ],
    )(refB3)

    grid = 10
    br = N // grid
    xn0, xn1 = pl.pallas_call(
        _norm_body,
        grid=(grid,),
        in_specs=[
            pl.BlockSpec((NC, br, DEG_W), lambda i: (0, i, 0)),
            pl.BlockSpec((br, D), lambda i: (i, 0)),
        ],
        out_specs=[
            pl.BlockSpec((br, DH), lambda i: (i, 0)),
            pl.BlockSpec((br, DH), lambda i: (i, 0)),
        ],
        out_shape=[
            jax.ShapeDtypeStruct((N, DH), jnp.float32),
            jax.ShapeDtypeStruct((N, DH), jnp.float32),
        ],
    )(degp, X)

    p0, p1 = pl.kernel(
        _conv_body,
        out_type=[
            jax.ShapeDtypeStruct((NC, n_pad, DH), jnp.float32),
            jax.ShapeDtypeStruct((NC, n_pad, DH), jnp.float32),
        ],
        mesh=mesh,
        compiler_params=pltpu.CompilerParams(use_tc_tiling_on_sc=False),
        scratch_types=[
            pltpu.VMEM((n_chunks, K), jnp.int32),
            pltpu.VMEM((n_chunks, K), jnp.int32),
            pltpu.VMEM((NB, K, DH), jnp.float32),
            pltpu.VMEM((128, DH), jnp.float32),
            pltpu.VMEM_SHARED((n_pad, DH), jnp.float32),
            pltpu.SemaphoreType.DMA((NB,)),
            pltpu.SemaphoreType.DMA((NB,)),
        ],
    )(xn0, xn1, refA3, refB3)

    b2 = b.reshape(1, U)
    out = pl.pallas_call(
        _out_body,
        grid=(grid,),
        in_specs=[
            pl.BlockSpec((br, D), lambda i: (i, 0)),
            pl.BlockSpec((NC, br, DH), lambda i: (0, i, 0)),
            pl.BlockSpec((NC, br, DH), lambda i: (0, i, 0)),
            pl.BlockSpec((NC, br, DEG_W), lambda i: (0, i, 0)),
            pl.BlockSpec((D, U), lambda i: (0, 0)),
            pl.BlockSpec((1, U), lambda i: (0, 0)),
        ],
        out_specs=pl.BlockSpec((br, U), lambda i: (i, 0)),
        out_shape=jax.ShapeDtypeStruct((N, U), jnp.float32),
    )(X, p0, p1, degp, W, b2)
    return out
